# Initial kernel scaffold; baseline (speedup 1.0000x reference)
import jax
import jax.numpy as jnp
from jax.experimental import pallas as pl


def kernel(x, table, W1, b1, W2, b2):
    raise NotImplementedError("write your pallas kernel here")


# speedup vs baseline: 16.0602x; 16.0602x over previous
import functools

import jax
import jax.numpy as jnp
from jax import lax
from jax.experimental import pallas as pl
from jax.experimental.pallas import tpu as pltpu
from jax.experimental.pallas import tpu_sc as plsc

VOCAB = 100000
EMBED = 64
HID = 16
OUT = 2
BATCH = 16384
HIST = 200
HALF = HIST // 2


def _mm_body(t_ref, w_ref, o_ref):
    o_ref[:] = jnp.dot(t_ref[:], w_ref[:], preferred_element_type=jnp.float32)


def _fold_table(table, W1):
    rows_blk = 2000
    grid = VOCAB // rows_blk
    return pl.pallas_call(
        _mm_body,
        grid=(grid,),
        in_specs=[
            pl.BlockSpec((rows_blk, EMBED), lambda i: (i, 0)),
            pl.BlockSpec((EMBED, HID), lambda i: (0, 0)),
        ],
        out_specs=pl.BlockSpec((rows_blk, HID), lambda i: (i, 0)),
        out_shape=jax.ShapeDtypeStruct((VOCAB, HID), jnp.float32),
    )(table, W1)


def _make_sc_pool():
    info = plsc.get_sparse_core_info()
    nc, ns = info.num_cores, info.num_subcores
    nw = nc * ns
    bpw = BATCH // nw
    ch = 128
    mesh = plsc.VectorSubcoreMesh(core_axis_name="c", subcore_axis_name="s")

    @functools.partial(
        pl.kernel,
        out_type=jax.ShapeDtypeStruct((BATCH, HID), jnp.float32),
        mesh=mesh,
        scratch_types=[
            pltpu.VMEM((ch, 2, HALF), jnp.int32),
            pltpu.VMEM((HALF, HID), jnp.float32),
            pltpu.VMEM((HALF, HID), jnp.float32),
            pltpu.VMEM((ch, HID), jnp.float32),
            pltpu.SemaphoreType.DMA,
        ],
        compiler_params=pltpu.CompilerParams(use_tc_tiling_on_sc=False),
    )
    def sc_pool(x_hbm, t1_hbm, out_hbm, idx_v, rows_a, rows_b, out_v, sem):
        wid = lax.axis_index("s") * nc + lax.axis_index("c")
        base = wid * bpw

        def chunk_body(ci, _):
            row0 = base + ci * ch
            pltpu.sync_copy(x_hbm.at[pl.ds(row0, ch)], idx_v)

            def row_body(r, _):
                cpa = pltpu.async_copy(t1_hbm.at[idx_v.at[r, 0]], rows_a, sem)
                cpb = pltpu.async_copy(t1_hbm.at[idx_v.at[r, 1]], rows_b, sem)
                cpa.wait()
                cpb.wait()

                def acc_body(j, accs):
                    a0, a1 = accs
                    return a0 + rows_a[j], a1 + rows_b[j]

                z = jnp.zeros((HID,), jnp.float32)
                a0, a1 = lax.fori_loop(0, HALF, acc_body, (z, z))
                out_v[r] = a0 + a1
                return 0

            lax.fori_loop(0, ch, row_body, 0)
            pltpu.sync_copy(out_v, out_hbm.at[pl.ds(row0, ch)])
            return 0

        lax.fori_loop(0, bpw // ch, chunk_body, 0)

    return sc_pool


def _head_body(s_ref, b1_ref, w2_ref, b2_ref, o_ref):
    h = jnp.maximum(s_ref[:] * (1.0 / HIST) + b1_ref[:], 0.0)
    o_ref[:] = jnp.dot(h, w2_ref[:], preferred_element_type=jnp.float32) + b2_ref[:]


def _head(sums, b1, W2, b2):
    rows_blk = 2048
    grid = BATCH // rows_blk
    return pl.pallas_call(
        _head_body,
        grid=(grid,),
        in_specs=[
            pl.BlockSpec((rows_blk, HID), lambda i: (i, 0)),
            pl.BlockSpec((1, HID), lambda i: (0, 0)),
            pl.BlockSpec((HID, OUT), lambda i: (0, 0)),
            pl.BlockSpec((1, OUT), lambda i: (0, 0)),
        ],
        out_specs=pl.BlockSpec((rows_blk, OUT), lambda i: (i, 0)),
        out_shape=jax.ShapeDtypeStruct((BATCH, OUT), jnp.float32),
    )(sums, b1.reshape(1, HID), W2, b2.reshape(1, OUT))


def kernel(x, table, W1, b1, W2, b2):
    table1 = _fold_table(table, W1)
    x2 = x.astype(jnp.int32).reshape(BATCH, 2, HALF)
    sums = _make_sc_pool()(x2, table1)
    return _head(sums, b1, W2, b2)


# speedup vs baseline: 33.9447x; 2.1136x over previous
import functools

import jax
import jax.numpy as jnp
from jax import lax
from jax.experimental import pallas as pl
from jax.experimental.pallas import tpu as pltpu
from jax.experimental.pallas import tpu_sc as plsc

VOCAB = 100000
EMBED = 64
HID = 16
OUT = 2
BATCH = 16384
HIST = 200
HALF = HIST // 2


def _mm_body(t_ref, w_ref, o_ref):
    o_ref[:] = jnp.dot(t_ref[:], w_ref[:], preferred_element_type=jnp.float32)


def _fold_table(table, W1):
    rows_blk = 2000
    grid = VOCAB // rows_blk
    return pl.pallas_call(
        _mm_body,
        grid=(grid,),
        in_specs=[
            pl.BlockSpec((rows_blk, EMBED), lambda i: (i, 0)),
            pl.BlockSpec((EMBED, HID), lambda i: (0, 0)),
        ],
        out_specs=pl.BlockSpec((rows_blk, HID), lambda i: (i, 0)),
        out_shape=jax.ShapeDtypeStruct((VOCAB, HID), jnp.float32),
    )(table, W1)


NBUF = 4


def _make_sc_pool():
    info = plsc.get_sparse_core_info()
    nc, ns = info.num_cores, info.num_subcores
    nw = nc * ns
    bpw = BATCH // nw
    mesh = plsc.VectorSubcoreMesh(core_axis_name="c", subcore_axis_name="s")

    @functools.partial(
        pl.kernel,
        out_type=jax.ShapeDtypeStruct((BATCH, HID), jnp.float32),
        mesh=mesh,
        scratch_types=[
            pltpu.VMEM((bpw, 2, HALF), jnp.int32),
            pltpu.VMEM((NBUF, 2, HALF, HID), jnp.float32),
            pltpu.VMEM((bpw, HID), jnp.float32),
            [pltpu.SemaphoreType.DMA] * NBUF,
        ],
        compiler_params=pltpu.CompilerParams(use_tc_tiling_on_sc=False),
    )
    def sc_pool(x_hbm, t1_hbm, out_hbm, idx_v, bufs, out_v, sems):
        wid = lax.axis_index("s") * nc + lax.axis_index("c")
        base = wid * bpw
        pltpu.sync_copy(x_hbm.at[pl.ds(base, bpw)], idx_v)

        def issue(slot, r):
            pltpu.async_copy(t1_hbm.at[idx_v.at[r, 0]], bufs.at[slot, 0], sems[slot])
            pltpu.async_copy(t1_hbm.at[idx_v.at[r, 1]], bufs.at[slot, 1], sems[slot])

        def drain(slot):
            pltpu.make_async_copy(t1_hbm.at[idx_v.at[0, 0]], bufs.at[slot, 0], sems[slot]).wait()
            pltpu.make_async_copy(t1_hbm.at[idx_v.at[0, 1]], bufs.at[slot, 1], sems[slot]).wait()

        for b in range(NBUF):
            issue(b, b)

        def outer(r0, _):
            for b in range(NBUF):
                r = r0 + b
                drain(b)
                accs = [jnp.zeros((HID,), jnp.float32)] * 8
                k = 0
                for half in range(2):
                    for j in range(HALF):
                        accs[k % 8] = accs[k % 8] + bufs[b, half, j]
                        k += 1
                out_v[r] = (
                    ((accs[0] + accs[1]) + (accs[2] + accs[3]))
                    + ((accs[4] + accs[5]) + (accs[6] + accs[7]))
                )

                @pl.when(r + NBUF < bpw)
                def _():
                    issue(b, r + NBUF)

            return 0

        lax.fori_loop(0, bpw // NBUF, lambda i, c: outer(i * NBUF, c), 0)
        pltpu.sync_copy(out_v, out_hbm.at[pl.ds(base, bpw)])

    return sc_pool


def _head_body(s_ref, b1_ref, w2_ref, b2_ref, o_ref):
    h = jnp.maximum(s_ref[:] * (1.0 / HIST) + b1_ref[:], 0.0)
    o_ref[:] = jnp.dot(h, w2_ref[:], preferred_element_type=jnp.float32) + b2_ref[:]


def _head(sums, b1, W2, b2):
    rows_blk = 2048
    grid = BATCH // rows_blk
    return pl.pallas_call(
        _head_body,
        grid=(grid,),
        in_specs=[
            pl.BlockSpec((rows_blk, HID), lambda i: (i, 0)),
            pl.BlockSpec((1, HID), lambda i: (0, 0)),
            pl.BlockSpec((HID, OUT), lambda i: (0, 0)),
            pl.BlockSpec((1, OUT), lambda i: (0, 0)),
        ],
        out_specs=pl.BlockSpec((rows_blk, OUT), lambda i: (i, 0)),
        out_shape=jax.ShapeDtypeStruct((BATCH, OUT), jnp.float32),
    )(sums, b1.reshape(1, HID), W2, b2.reshape(1, OUT))


def kernel(x, table, W1, b1, W2, b2):
    table1 = _fold_table(table, W1)
    x2 = x.astype(jnp.int32).reshape(BATCH, 2, HALF)
    sums = _make_sc_pool()(x2, table1)
    return _head(sums, b1, W2, b2)


# speedup vs baseline: 43.5978x; 1.2844x over previous
import functools

import jax
import jax.numpy as jnp
from jax import lax
from jax.experimental import pallas as pl
from jax.experimental.pallas import tpu as pltpu
from jax.experimental.pallas import tpu_sc as plsc

VOCAB = 100000
EMBED = 64
HID = 16
OUT = 2
BATCH = 16384
HIST = 200
CHUNK_A = 104
CHUNK_B = 96
PACK = 128 // HID


def _mm_body(t_ref, w_ref, o_ref):
    o_ref[:] = jnp.dot(t_ref[0], w_ref[:], preferred_element_type=jnp.float32)[None]


def _fold_table(table, W1):
    grid = 50
    pk_rows = VOCAB // PACK // grid
    t8 = table.reshape(grid, pk_rows, PACK * EMBED)
    w1big = jnp.kron(jnp.eye(PACK, dtype=W1.dtype), W1)
    packed = pl.pallas_call(
        _mm_body,
        grid=(grid,),
        in_specs=[
            pl.BlockSpec((1, pk_rows, PACK * EMBED), lambda i: (i, 0, 0)),
            pl.BlockSpec((PACK * EMBED, PACK * HID), lambda i: (0, 0)),
        ],
        out_specs=pl.BlockSpec((1, pk_rows, PACK * HID), lambda i: (i, 0, 0)),
        out_shape=jax.ShapeDtypeStruct((grid, pk_rows, PACK * HID), jnp.float32),
    )(t8, w1big)
    return packed.reshape(VOCAB, HID)


NBUF = 4


def _make_sc_pool():
    info = plsc.get_sparse_core_info()
    nc, ns = info.num_cores, info.num_subcores
    nw = nc * ns
    bpw = BATCH // nw
    mesh = plsc.VectorSubcoreMesh(core_axis_name="c", subcore_axis_name="s")

    @functools.partial(
        pl.kernel,
        out_type=jax.ShapeDtypeStruct((BATCH, HID), jnp.float32),
        mesh=mesh,
        scratch_types=[
            pltpu.VMEM((bpw, HIST), jnp.int32),
            pltpu.VMEM((NBUF, HIST, HID), jnp.float32),
            pltpu.VMEM((bpw, HID), jnp.float32),
            [pltpu.SemaphoreType.DMA] * NBUF,
        ],
        compiler_params=pltpu.CompilerParams(use_tc_tiling_on_sc=False),
    )
    def sc_pool(x_hbm, t1_hbm, out_hbm, idx_v, bufs, out_v, sems):
        wid = lax.axis_index("s") * nc + lax.axis_index("c")
        base = wid * bpw
        pltpu.sync_copy(x_hbm.at[pl.ds(base, bpw)], idx_v)

        def issue(slot, r):
            pltpu.async_copy(
                t1_hbm.at[idx_v.at[r, pl.ds(0, CHUNK_A)]],
                bufs.at[slot, pl.ds(0, CHUNK_A)],
                sems[slot],
            )
            pltpu.async_copy(
                t1_hbm.at[idx_v.at[r, pl.ds(CHUNK_A, CHUNK_B)]],
                bufs.at[slot, pl.ds(CHUNK_A, CHUNK_B)],
                sems[slot],
            )

        def drain(slot):
            pltpu.make_async_copy(
                t1_hbm.at[idx_v.at[0, pl.ds(0, CHUNK_A)]],
                bufs.at[slot, pl.ds(0, CHUNK_A)],
                sems[slot],
            ).wait()
            pltpu.make_async_copy(
                t1_hbm.at[idx_v.at[0, pl.ds(CHUNK_A, CHUNK_B)]],
                bufs.at[slot, pl.ds(CHUNK_A, CHUNK_B)],
                sems[slot],
            ).wait()

        for b in range(NBUF):
            issue(b, b)

        def outer(r0, _):
            for b in range(NBUF):
                r = r0 + b
                drain(b)
                accs = [jnp.zeros((HID,), jnp.float32)] * 8
                for j in range(HIST):
                    accs[j % 8] = accs[j % 8] + bufs[b, j]
                out_v[r] = (
                    ((accs[0] + accs[1]) + (accs[2] + accs[3]))
                    + ((accs[4] + accs[5]) + (accs[6] + accs[7]))
                )

                @pl.when(r + NBUF < bpw)
                def _():
                    issue(b, r + NBUF)

            return 0

        lax.fori_loop(0, bpw // NBUF, lambda i, c: outer(i * NBUF, c), 0)
        pltpu.sync_copy(out_v, out_hbm.at[pl.ds(base, bpw)])

    return sc_pool


def _head_body(s_ref, b1_ref, w2_ref, b2_ref, o_ref):
    h = jnp.maximum(s_ref[:] * (1.0 / HIST) + b1_ref[:], 0.0)
    o_ref[:] = jnp.dot(h, w2_ref[:], preferred_element_type=jnp.float32) + b2_ref[:]


def _head(sums, b1, W2, b2):
    rows_blk = 2048
    grid = BATCH // rows_blk
    return pl.pallas_call(
        _head_body,
        grid=(grid,),
        in_specs=[
            pl.BlockSpec((rows_blk, HID), lambda i: (i, 0)),
            pl.BlockSpec((1, HID), lambda i: (0, 0)),
            pl.BlockSpec((HID, OUT), lambda i: (0, 0)),
            pl.BlockSpec((1, OUT), lambda i: (0, 0)),
        ],
        out_specs=pl.BlockSpec((rows_blk, OUT), lambda i: (i, 0)),
        out_shape=jax.ShapeDtypeStruct((BATCH, OUT), jnp.float32),
    )(sums, b1.reshape(1, HID), W2, b2.reshape(1, OUT))


def kernel(x, table, W1, b1, W2, b2):
    table1 = _fold_table(table, W1)
    sums = _make_sc_pool()(x.astype(jnp.int32), table1)
    return _head(sums, b1, W2, b2)
